# baseline (speedup 1.0000x reference)
import jax
import jax.numpy as jnp
from jax.experimental import pallas as pl

_NUM_E = 512
_DIM = 64
_AUG = 72
_G = 8


def _vq_body(x_ref, e_ref, eh_ref, el_ref, o_ref):
    ee = e_ref[...]
    es = jnp.sum(ee * ee, axis=1)[None, :] * 0.25
    for g in range(_G):
        xb = x_ref[g]

        prod2 = jax.lax.dot_general(
            xb, ee, (((0,), (1,)), ((), ())),
            preferred_element_type=jnp.float32)
        zs = jnp.sum(xb * xb, axis=0)[:, None]
        dist = (zs + es) + prod2

        m = jnp.min(dist, axis=1, keepdims=True)
        onehot = (dist == m).astype(jnp.bfloat16)

        res = (jax.lax.dot_general(
                   eh_ref[...], onehot, (((0,), (1,)), ((), ())),
                   preferred_element_type=jnp.float32)
               + jax.lax.dot_general(
                   el_ref[...], onehot, (((0,), (1,)), ((), ())),
                   preferred_element_type=jnp.float32))
        o_ref[g] = res[0:_DIM] * (1.0 / res[_DIM:_DIM + 1])


def kernel(x, e):
    B, C, H, W = x.shape
    P = H * W
    xr = x.reshape(B, C, P)
    e2 = -2.0 * e
    e_hi = e.astype(jnp.bfloat16)
    e_lo = (e - e_hi.astype(jnp.float32)).astype(jnp.bfloat16)
    ones_col = jnp.ones((_NUM_E, 1), jnp.bfloat16)
    pad = jnp.zeros((_NUM_E, _AUG - _DIM - 1), jnp.bfloat16)
    eh_aug = jnp.concatenate([e_hi, ones_col, pad], axis=1)
    el_aug = jnp.concatenate([e_lo, jnp.zeros((_NUM_E, _AUG - _DIM), jnp.bfloat16)], axis=1)
    out = pl.pallas_call(
        _vq_body,
        grid=(B // _G,),
        in_specs=[
            pl.BlockSpec((_G, C, P), lambda i: (i, 0, 0)),
            pl.BlockSpec((_NUM_E, _DIM), lambda i: (0, 0)),
            pl.BlockSpec((_NUM_E, _AUG), lambda i: (0, 0)),
            pl.BlockSpec((_NUM_E, _AUG), lambda i: (0, 0)),
        ],
        out_specs=pl.BlockSpec((_G, C, P), lambda i: (i, 0, 0)),
        out_shape=jax.ShapeDtypeStruct((B, C, P), jnp.float32),
    )(xr, e2, eh_aug, el_aug)
    return out.reshape(B, C, H, W)


# speedup vs baseline: 1.1178x; 1.1178x over previous
import jax
import jax.numpy as jnp
from jax.experimental import pallas as pl

_NUM_E = 512
_DIM = 64
_AUG = 72
_G = 4


def _vq_body(x_ref, e_ref, eh_ref, el_ref, o_ref):
    ee = e_ref[...]
    es = jnp.sum(ee * ee, axis=1)[None, :] * 0.25
    for g in range(_G):
        xb = x_ref[g]

        prod2 = jax.lax.dot_general(
            xb, ee, (((0,), (1,)), ((), ())),
            preferred_element_type=jnp.float32)
        zs = jnp.sum(xb * xb, axis=0)[:, None]
        dist = (zs + es) + prod2

        m = jnp.min(dist, axis=1, keepdims=True)
        onehot = (dist == m).astype(jnp.bfloat16)

        res = jax.lax.dot_general(
            eh_ref[...], onehot, (((0,), (1,)), ((), ())),
            preferred_element_type=jnp.float32)
        o_ref[g] = res[0:_DIM] * (1.0 / res[_DIM:_DIM + 1])


def kernel(x, e):
    B, C, H, W = x.shape
    P = H * W
    xr = x.reshape(B, C, P)
    e2 = -2.0 * e
    e_hi = e.astype(jnp.bfloat16)
    e_lo = (e - e_hi.astype(jnp.float32)).astype(jnp.bfloat16)
    ones_col = jnp.ones((_NUM_E, 1), jnp.bfloat16)
    pad = jnp.zeros((_NUM_E, _AUG - _DIM - 1), jnp.bfloat16)
    eh_aug = jnp.concatenate([e_hi, ones_col, pad], axis=1)
    el_aug = jnp.concatenate([e_lo, jnp.zeros((_NUM_E, _AUG - _DIM), jnp.bfloat16)], axis=1)
    out = pl.pallas_call(
        _vq_body,
        grid=(B // _G,),
        in_specs=[
            pl.BlockSpec((_G, C, P), lambda i: (i, 0, 0)),
            pl.BlockSpec((_NUM_E, _DIM), lambda i: (0, 0)),
            pl.BlockSpec((_NUM_E, _AUG), lambda i: (0, 0)),
            pl.BlockSpec((_NUM_E, _AUG), lambda i: (0, 0)),
        ],
        out_specs=pl.BlockSpec((_G, C, P), lambda i: (i, 0, 0)),
        out_shape=jax.ShapeDtypeStruct((B, C, P), jnp.float32),
    )(xr, e2, eh_aug, el_aug)
    return out.reshape(B, C, H, W)


# speedup vs baseline: 1.1199x; 1.0018x over previous
import jax
import jax.numpy as jnp
from jax.experimental import pallas as pl
from jax.experimental.pallas import tpu as pltpu

_NUM_E = 512
_DIM = 64
_AUG = 72
_G = 4


def _vq_body(x_ref, e_ref, eh_ref, el_ref, o_ref):
    ee = e_ref[...]
    es = jnp.sum(ee * ee, axis=1)[None, :] * 0.25
    for g in range(_G):
        xb = x_ref[g]

        prod2 = jax.lax.dot_general(
            xb, ee, (((0,), (1,)), ((), ())),
            preferred_element_type=jnp.float32)
        zs = jnp.sum(xb * xb, axis=0)[:, None]
        dist = (zs + es) + prod2

        m = jnp.min(dist, axis=1, keepdims=True)
        onehot = (dist == m).astype(jnp.bfloat16)

        res = jax.lax.dot_general(
            eh_ref[...], onehot, (((0,), (1,)), ((), ())),
            preferred_element_type=jnp.float32)
        o_ref[g] = res[0:_DIM] * (1.0 / res[_DIM:_DIM + 1])


def kernel(x, e):
    B, C, H, W = x.shape
    P = H * W
    xr = x.reshape(B, C, P)
    e2 = -2.0 * e
    e_hi = e.astype(jnp.bfloat16)
    e_lo = (e - e_hi.astype(jnp.float32)).astype(jnp.bfloat16)
    ones_col = jnp.ones((_NUM_E, 1), jnp.bfloat16)
    pad = jnp.zeros((_NUM_E, _AUG - _DIM - 1), jnp.bfloat16)
    eh_aug = jnp.concatenate([e_hi, ones_col, pad], axis=1)
    el_aug = jnp.concatenate([e_lo, jnp.zeros((_NUM_E, _AUG - _DIM), jnp.bfloat16)], axis=1)
    out = pl.pallas_call(
        _vq_body,
        grid=(B // _G,),
        in_specs=[
            pl.BlockSpec((_G, C, P), lambda i: (i, 0, 0)),
            pl.BlockSpec((_NUM_E, _DIM), lambda i: (0, 0)),
            pl.BlockSpec((_NUM_E, _AUG), lambda i: (0, 0)),
            pl.BlockSpec((_NUM_E, _AUG), lambda i: (0, 0)),
        ],
        out_specs=pl.BlockSpec((_G, C, P), lambda i: (i, 0, 0)),
        out_shape=jax.ShapeDtypeStruct((B, C, P), jnp.float32),
        compiler_params=pltpu.CompilerParams(dimension_semantics=("parallel",)),
    )(xr, e2, eh_aug, el_aug)
    return out.reshape(B, C, H, W)
